# baseline (speedup 1.0000x reference)
import jax
import jax.numpy as jnp
import numpy as np
from jax import lax
from jax.experimental import pallas as pl
from jax.experimental.pallas import tpu as pltpu
from jax.experimental.pallas import tpu_sc as plsc

ROWS = 128
COLS = 32768
K_OUT = 64
L = 16
NVREG = COLS // L
NC = 2
NS = 16
NW = NC * NS
RPW = ROWS // NW
CAP = 4096
U0 = 8
U1 = 8

_SIGN = np.uint32(0x80000000)
_LOW = np.uint32(0x7FFFFFFF)


def _to_key(bits):
  sign = bits >> 31
  return bits ^ ((sign * _LOW) | _SIGN)


def _from_key(key):
  sign = key >> 31
  return key ^ (((np.uint32(1) - sign) * _LOW) | _SIGN)


def _lane_iota():
  return lax.iota(jnp.int32, L)


def _perm(x, perm):
  dnums = lax.GatherDimensionNumbers(
      offset_dims=(), collapsed_slice_dims=(0,), start_index_map=(0,))
  return lax.gather(x, perm[:, None], dnums, slice_sizes=(1,),
                    mode=lax.GatherScatterMode.PROMISE_IN_BOUNDS)


def _clean_desc16(x):
  lane = _lane_iota()
  for k in (8, 4, 2, 1):
    p = _perm(x, lane ^ k)
    hi = jnp.maximum(x, p)
    lo = jnp.minimum(x, p)
    x = jnp.where((lane & k) == 0, hi, lo)
  return x


def _merge32(a, b):
  rb = lax.rev(b, (0,))
  return _clean_desc16(jnp.maximum(a, rb)), _clean_desc16(jnp.minimum(a, rb))


def _merge64(a0, a1, b0, b1):
  rb0 = lax.rev(b1, (0,))
  rb1 = lax.rev(b0, (0,))
  h0, h1 = jnp.maximum(a0, rb0), jnp.maximum(a1, rb1)
  l0, l1 = jnp.minimum(a0, rb0), jnp.minimum(a1, rb1)
  t0 = _clean_desc16(jnp.maximum(h0, h1))
  t1 = _clean_desc16(jnp.minimum(h0, h1))
  u0 = _clean_desc16(jnp.maximum(l0, l1))
  u1 = _clean_desc16(jnp.minimum(l0, l1))
  return t0, t1, u0, u1


def _scalar(x):
  return x[0]


def _walk(hist_ref, k_rem, ci0=15):
  def cond(c):
    ci, cum, found, p, cg, cp = c
    return jnp.logical_and(jnp.logical_not(found), ci >= 0)

  def body(c):
    ci, cum, found, p, cg, cp = c
    v = hist_ref[pl.ds(ci * L, L)]
    rv = lax.rev(v, (0,))
    cs = plsc.cumsum(rv)
    tot = cs[L - 1]
    hit = (cum + tot) >= k_rem
    crossed = (cum + cs) >= k_rem
    jj = _scalar(plsc.all_reduce_ffs(crossed))
    excl = cs - rv
    lane = _lane_iota()
    at_jj = lane == jj
    cg_here = cum + jnp.sum(jnp.where(at_jj, excl, 0))
    cp_here = jnp.sum(jnp.where(at_jj, rv, 0))
    p_here = ci * L + (L - 1 - jj)
    ci2 = jnp.where(hit, ci, ci - 1)
    cum2 = jnp.where(hit, cum, cum + tot)
    p2 = jnp.where(hit, p_here, p)
    cg2 = jnp.where(hit, cg_here, cg)
    cp2 = jnp.where(hit, cp_here, cp)
    return ci2, cum2, hit, p2, cg2, cp2

  zero = np.int32(0)
  ci, cum, found, p, cg, cp = lax.while_loop(
      cond, body, (np.int32(ci0), zero, False, zero, zero, zero))
  return p, cg, cp


def _walk_vec(v, k_rem):
  lane = _lane_iota()
  rv = lax.rev(v, (0,))
  cs = plsc.cumsum(rv)
  crossed = cs >= k_rem
  jj = _scalar(plsc.all_reduce_ffs(crossed))
  excl = cs - rv
  at_jj = lane == jj
  cg = jnp.sum(jnp.where(at_jj, excl, 0))
  cp = jnp.sum(jnp.where(at_jj, rv, 0))
  return L - 1 - jj, cg, cp


def _clear_hist(hist_ref):
  zeros = jnp.full((L,), 0, jnp.int32)
  for i in range(256 // L):
    hist_ref[pl.ds(i * L, L)] = zeros


def _scatter_append(ref, base_vec, key, mask):
  pos = base_vec + plsc.cumsum(mask.astype(jnp.int32)) - 1
  plsc.store_scatter(ref, [pos], plsc.bitcast(key, jnp.int32), mask=mask)
  return base_vec + plsc.all_reduce_population_count(mask)


def _process_row(buf, keybuf, hist, winners, canda, candb, candc, outbuf,
                 out_base):
  lane = _lane_iota()
  ones_u32 = jnp.full((L,), 1, jnp.uint32)
  zeros_u32 = jnp.full((L,), 0, jnp.uint32)

  def pass0(i, accs):
    base = i * L
    packed = zeros_u32
    for u in range(U0):
      x = buf[pl.ds(base + u * L, L)]
      key = _to_key(plsc.bitcast(x, jnp.uint32))
      keybuf[pl.ds(base + u * L, L)] = key
      sh = (key >> 27) & np.uint32(0x1C)
      packed = packed + (ones_u32 << sh)
    new = []
    for t in range(8):
      new.append(accs[t] + ((packed >> (4 * t)) & np.uint32(0xF)))
    return tuple(new)

  accs = plsc.parallel_loop(
      0, NVREG, step=U0, unroll=2, carry=(zeros_u32,) * 8)(pass0)

  tot = jnp.full((L,), 0, jnp.int32)
  for t in range(8):
    s = jnp.sum(accs[t].astype(jnp.int32))
    tot = jnp.where(lane == t, s, tot)
  p0, cg0, cp0 = _walk_vec(tot, np.int32(K_OUT))

  k_rem = np.int32(K_OUT) - cg0
  src_cnt0 = cg0 + cp0
  src_fits = src_cnt0 <= CAP
  prefix = p0.astype(jnp.uint32)
  thr0 = prefix << 29

  zeros_i32 = jnp.full((L,), 0, jnp.int32)

  def pass1(i, ccnt):
    base = i * L
    for u in range(U1):
      key = keybuf[pl.ds(base + u * L, L)]
      m_c = jnp.logical_and(key >= thr0, src_fits)
      pc = plsc.all_reduce_population_count(m_c)

      @pl.when(pc[0] > 0)
      def _(key=key, m_c=m_c, ccnt=ccnt):
        pos = ccnt + plsc.cumsum(m_c.astype(jnp.int32)) - 1
        plsc.store_scatter(canda, [pos], plsc.bitcast(key, jnp.int32),
                           mask=m_c)

      ccnt = ccnt + pc
    return ccnt

  plsc.parallel_loop(0, NVREG, step=U1, unroll=2, carry=zeros_i32)(pass1)
  wcnt = zeros_i32

  def hist_narrow(src_ref, src_cnt, shift, width, prefix_cur):
    mask_b = np.uint32((1 << width) - 1)

    def body(i, c):
      key = plsc.bitcast(src_ref[pl.ds(i * L, L)], jnp.uint32)
      valid = jnp.logical_and(
          (i * L + lane) < src_cnt,
          (key >> (shift + width)) == prefix_cur)
      b = ((key >> shift) & mask_b).astype(jnp.int32)
      cnt, last = plsc.scan_count(b, mask=valid)
      plsc.addupdate_scatter(hist, [b], cnt,
                             mask=jnp.logical_and(last, valid))
      return c

    return body

  def hist_row(shift, width, prefix_cur):
    mask_b = np.uint32((1 << width) - 1)

    def body(i, c):
      key = keybuf[pl.ds(i * L, L)]
      m_pre = (key >> (shift + width)) == prefix_cur
      b = ((key >> shift) & mask_b).astype(jnp.int32)
      cnt, last = plsc.scan_count(b, mask=m_pre)
      plsc.addupdate_scatter(hist, [b], cnt,
                             mask=jnp.logical_and(last, m_pre))
      return c

    return body

  def split_narrow(src_ref, src_cnt, dst_ref, shift, prefix_next):
    def body(i, carry):
      wcnt, ccnt = carry
      key = plsc.bitcast(src_ref[pl.ds(i * L, L)], jnp.uint32)
      valid = (i * L + lane) < src_cnt
      sk = key >> shift
      m_gt = jnp.logical_and(valid, sk > prefix_next)
      wcnt = _scatter_append(winners, wcnt, key, m_gt)
      if dst_ref is not None:
        m_eq = jnp.logical_and(valid, sk == prefix_next)
        ccnt = _scatter_append(dst_ref, ccnt, key, m_eq)
      return wcnt, ccnt

    return body

  def split_row(dst_ref, dst_fits, shift, width, prefix_cur, prefix_next,
                restrict_gt):
    def body(i, carry):
      wcnt, ccnt = carry
      key = keybuf[pl.ds(i * L, L)]
      sk = key >> shift
      m_gt = sk > prefix_next
      if restrict_gt:
        m_pre = (key >> (shift + width)) == prefix_cur
        m_gt = jnp.logical_and(m_pre, m_gt)
      wcnt = _scatter_append(winners, wcnt, key, m_gt)
      if dst_ref is not None:
        m_eq = sk == prefix_next
        ccnt = _scatter_append(dst_ref, ccnt, key,
                               jnp.logical_and(m_eq, dst_fits))
      return wcnt, ccnt

    return body

  src_ref, src_cnt = canda, src_cnt0
  for shift, width, dst_ref, restrict_gt in (
      (21, 8, candb, False), (13, 8, candc, True),
      (5, 8, canda, True), (0, 5, None, True)):
    n_narrow = jnp.where(src_fits, (src_cnt + L - 1) // L, 0)
    n_row = jnp.where(src_fits, 0, NVREG)
    _clear_hist(hist)
    lax.fori_loop(
        0, n_narrow, hist_narrow(src_ref, src_cnt, shift, width, prefix), 0)
    lax.fori_loop(0, n_row, hist_row(shift, width, prefix), 0)
    p, cg, cp = _walk(hist, k_rem, ci0=(15 if width == 8 else 1))
    dst_fits = cp <= CAP
    prefix_next = (prefix << width) | p.astype(jnp.uint32)
    wcnt, ccnt = lax.fori_loop(
        0, n_narrow, split_narrow(src_ref, src_cnt, dst_ref, shift,
                                  prefix_next),
        (wcnt, zeros_i32))
    wcnt, ccnt = lax.fori_loop(
        0, n_row, split_row(dst_ref, dst_fits, shift, width, prefix,
                            prefix_next, restrict_gt),
        (wcnt, ccnt))
    prefix = prefix_next
    k_rem = k_rem - cg
    src_ref, src_cnt, src_fits = dst_ref, cp, dst_fits

  v64 = prefix

  v64_i32 = plsc.bitcast(jnp.full((L,), v64, jnp.uint32), jnp.int32)
  for t in range(4):
    off = lane + t * L
    plsc.store_scatter(winners, [wcnt + off], v64_i32, mask=off < k_rem)

  w = [plsc.bitcast(winners[pl.ds(t * L, L)], jnp.uint32) for t in range(4)]
  s16 = [lax.rev(lax.sort(w[t], dimension=0), (0,)) for t in range(4)]
  a0, a1 = _merge32(s16[0], s16[1])
  b0, b1 = _merge32(s16[2], s16[3])
  o = _merge64(a0, a1, b0, b1)
  for t in range(4):
    outbuf[pl.ds(out_base + t * L, L)] = plsc.bitcast(
        _from_key(o[t]), jnp.float32)


def _body(in_hbm, out_hbm, rowa, rowb, keybuf, hist, winners, canda, candb,
          candc, outbuf, sem_a, sem_b):
  wid = lax.axis_index("s") * NC + lax.axis_index("c")
  base_row = wid * RPW

  bufs = (rowa, rowb)
  sems = (sem_a, sem_b)
  pltpu.make_async_copy(in_hbm.at[base_row], rowa, sem_a).start()
  for j in range(RPW):
    buf = bufs[j % 2]
    sem = sems[j % 2]
    pltpu.make_async_copy(in_hbm.at[base_row + j], buf, sem).wait()
    if j + 1 < RPW:
      pltpu.make_async_copy(
          in_hbm.at[base_row + j + 1], bufs[(j + 1) % 2], sems[(j + 1) % 2]
      ).start()
    _process_row(buf, keybuf, hist, winners, canda, candb, candc, outbuf,
                 j * K_OUT)
  pltpu.sync_copy(outbuf, out_hbm.at[pl.ds(wid * (RPW * K_OUT), RPW * K_OUT)])


def _make_kernel():
  mesh = plsc.VectorSubcoreMesh(core_axis_name="c", subcore_axis_name="s")
  return pl.kernel(
      _body,
      out_type=jax.ShapeDtypeStruct((ROWS * K_OUT,), jnp.float32),
      mesh=mesh,
      scratch_types=[
          pltpu.VMEM((COLS,), jnp.float32),
          pltpu.VMEM((COLS,), jnp.float32),
          pltpu.VMEM((COLS,), jnp.uint32),
          pltpu.VMEM((256,), jnp.int32),
          pltpu.VMEM((128,), jnp.int32),
          pltpu.VMEM((CAP,), jnp.int32),
          pltpu.VMEM((CAP,), jnp.int32),
          pltpu.VMEM((CAP,), jnp.int32),
          pltpu.VMEM((RPW * K_OUT,), jnp.float32),
          pltpu.SemaphoreType.DMA,
          pltpu.SemaphoreType.DMA,
      ],
      compiler_params=pltpu.CompilerParams(needs_layout_passes=False),
  )


@jax.jit
def kernel(inputs):
  return _make_kernel()(inputs).reshape(ROWS, K_OUT)
